# Initial kernel scaffold; baseline (speedup 1.0000x reference)
import jax
import jax.numpy as jnp
from jax.experimental import pallas as pl


def kernel(h, W_fc, b_fc, Wa, ba, Wb, bb, Wc, bc, Wcls, bcls, Wi, bi):
    raise NotImplementedError("write your pallas kernel here")


# speedup vs baseline: 1.1722x; 1.1722x over previous
import functools

import jax
import jax.numpy as jnp
from jax.experimental import pallas as pl
from jax.experimental.pallas import tpu as pltpu

N = 100000
D_IN = 768
D_HID = 512
D_ATT = 256
K = 4
TILE = 2000
NEG = float("-inf")
POS = float("inf")
BIG = 2**30


def _tile_topk(A, gidx, h1, largest: bool):
    T = A.shape[0]
    Acur = A
    vals = []
    sels = []
    for _ in range(K):
        if largest:
            v = jnp.max(Acur)
        else:
            v = jnp.min(Acur)
        eq = Acur == v
        sel = jnp.min(jnp.where(eq, gidx, BIG))
        one = gidx == sel
        Acur = jnp.where(one, NEG if largest else POS, Acur)
        vals.append(v)
        sels.append(sel)
    tv = jnp.stack(vals).reshape(K, 1)
    ti = jnp.stack(sels).reshape(K, 1)
    row_idx = gidx[0, 0] + jax.lax.broadcasted_iota(jnp.int32, (K, T), 1)
    onehot = (row_idx == ti).astype(jnp.bfloat16)
    rows = jnp.dot(onehot, h1, preferred_element_type=jnp.float32)
    return tv, ti, rows


def _merge_topk(rv, ri, rr, tv, ti, tr, largest: bool):
    cv = jnp.concatenate([rv, tv], axis=0)
    ci = jnp.concatenate([ri, ti], axis=0)
    cr = jnp.concatenate([rr, tr], axis=0)
    nv, ni, nr = [], [], []
    for _ in range(K):
        if largest:
            v = jnp.max(cv)
        else:
            v = jnp.min(cv)
        eq = cv == v
        sel = jnp.min(jnp.where(eq, ci, BIG))
        one = (ci == sel) & eq
        row = jnp.sum(jnp.where(one, cr, 0.0), axis=0, keepdims=True)
        cv = jnp.where(one, NEG if largest else POS, cv)
        nv.append(v)
        ni.append(sel)
        nr.append(row)
    return (jnp.stack(nv).reshape(K, 1), jnp.stack(ni).reshape(K, 1),
            jnp.concatenate(nr, axis=0))


def _chief_kernel(h_ref, Wfc_ref, bfc_ref, Wa_ref, ba_ref, Wb_ref, bb_ref,
                  Wc_ref, bc_ref, Wcls_ref, bcls_ref, Wi_ref, bi_ref,
                  out_ref,
                  m_ref, s_ref, acc_ref,
                  tv_ref, ti_ref, tr_ref,
                  bv_ref, bi_idx_ref, br_ref):
    i = pl.program_id(0)
    nsteps = pl.num_programs(0)

    @pl.when(i == 0)
    def _init():
        m_ref[0, 0] = jnp.float32(NEG)
        s_ref[0, 0] = jnp.float32(0.0)
        acc_ref[...] = jnp.zeros_like(acc_ref)
        tv_ref[...] = jnp.full_like(tv_ref, NEG)
        ti_ref[...] = jnp.full_like(ti_ref, BIG)
        tr_ref[...] = jnp.zeros_like(tr_ref)
        bv_ref[...] = jnp.full_like(bv_ref, POS)
        bi_idx_ref[...] = jnp.full_like(bi_idx_ref, BIG)
        br_ref[...] = jnp.zeros_like(br_ref)

    h16 = h_ref[...].astype(jnp.bfloat16)
    h1 = jnp.maximum(
        jnp.dot(h16, Wfc_ref[...], preferred_element_type=jnp.float32)
        + bfc_ref[...], 0.0)
    h1_16 = h1.astype(jnp.bfloat16)
    a = jnp.tanh(
        jnp.dot(h1_16, Wa_ref[...], preferred_element_type=jnp.float32)
        + ba_ref[...])
    b = jax.nn.sigmoid(
        jnp.dot(h1_16, Wb_ref[...], preferred_element_type=jnp.float32)
        + bb_ref[...])
    ab = (a * b).astype(jnp.bfloat16).astype(jnp.float32)
    A = jnp.sum(ab * Wc_ref[...], axis=1, keepdims=True) + bc_ref[0, 0]

    T = h16.shape[0]
    gidx = i * T + jax.lax.broadcasted_iota(jnp.int32, (T, 1), 0)

    tv, ti, tr = _tile_topk(A, gidx, h1_16, largest=True)
    nv, ni, nr = _merge_topk(tv_ref[...], ti_ref[...], tr_ref[...],
                             tv, ti, tr, largest=True)
    tv_ref[...], ti_ref[...], tr_ref[...] = nv, ni, nr

    lv, li, lr = _tile_topk(A, gidx, h1_16, largest=False)
    mv, mi, mr = _merge_topk(bv_ref[...], bi_idx_ref[...], br_ref[...],
                             lv, li, lr, largest=False)
    bv_ref[...], bi_idx_ref[...], br_ref[...] = mv, mi, mr

    m_old = m_ref[0, 0]
    m_new = jnp.maximum(m_old, tv[0, 0])
    corr = jnp.exp(m_old - m_new)
    w = jnp.exp(A - m_new)
    s_ref[0, 0] = s_ref[0, 0] * corr + jnp.sum(w)
    acc_ref[...] = acc_ref[...] * corr + jnp.sum(w * h1, axis=0,
                                                 keepdims=True)
    m_ref[0, 0] = m_new

    @pl.when(i == nsteps - 1)
    def _epilogue():
        M = (acc_ref[...] / s_ref[0, 0]).astype(jnp.bfloat16)
        bag = jnp.dot(M, Wcls_ref[...],
                      preferred_element_type=jnp.float32) + bcls_ref[...]
        allr = jnp.concatenate([tr_ref[...], br_ref[...]],
                               axis=0).astype(jnp.bfloat16)
        inst = jnp.dot(allr, Wi_ref[...],
                       preferred_element_type=jnp.float32) + bi_ref[...]
        out_ref[...] = jnp.zeros((16, 128), dtype=jnp.float32)
        out_ref[0:1, 0:2] = bag
        out_ref[1:1 + 2 * K, 0:2] = inst


@functools.partial(jax.jit, static_argnames=())
def kernel(h, W_fc, b_fc, Wa, ba, Wb, bb, Wc, bc, Wcls, bcls, Wi, bi):
    nsteps = N // TILE
    const = lambda *_: (0, 0)
    out = pl.pallas_call(
        _chief_kernel,
        grid=(nsteps,),
        in_specs=[
            pl.BlockSpec((TILE, D_IN), lambda i: (i, 0)),
            pl.BlockSpec((D_IN, D_HID), const),
            pl.BlockSpec((1, D_HID), const),
            pl.BlockSpec((D_HID, D_ATT), const),
            pl.BlockSpec((1, D_ATT), const),
            pl.BlockSpec((D_HID, D_ATT), const),
            pl.BlockSpec((1, D_ATT), const),
            pl.BlockSpec((1, D_ATT), const),
            pl.BlockSpec((1, 1), const),
            pl.BlockSpec((D_HID, 2), const),
            pl.BlockSpec((1, 2), const),
            pl.BlockSpec((D_HID, 2), const),
            pl.BlockSpec((1, 2), const),
        ],
        out_specs=pl.BlockSpec((16, 128), const),
        out_shape=jax.ShapeDtypeStruct((16, 128), jnp.float32),
        scratch_shapes=[
            pltpu.SMEM((1, 1), jnp.float32),
            pltpu.SMEM((1, 1), jnp.float32),
            pltpu.VMEM((1, D_HID), jnp.float32),
            pltpu.VMEM((K, 1), jnp.float32),
            pltpu.VMEM((K, 1), jnp.int32),
            pltpu.VMEM((K, D_HID), jnp.float32),
            pltpu.VMEM((K, 1), jnp.float32),
            pltpu.VMEM((K, 1), jnp.int32),
            pltpu.VMEM((K, D_HID), jnp.float32),
        ],
    )(
        h, W_fc.astype(jnp.bfloat16), b_fc.reshape(1, D_HID),
        Wa.astype(jnp.bfloat16), ba.reshape(1, D_ATT),
        Wb.astype(jnp.bfloat16), bb.reshape(1, D_ATT),
        Wc.astype(jnp.bfloat16).astype(jnp.float32).reshape(1, D_ATT),
        bc.reshape(1, 1),
        Wcls.astype(jnp.bfloat16), bcls.reshape(1, 2),
        Wi.astype(jnp.bfloat16), bi.reshape(1, 2),
    )
    return out[0:1 + 2 * K, 0:2]
